# baseline (speedup 1.0000x reference)
import jax
import jax.numpy as jnp
from jax.experimental import pallas as pl
from jax.experimental.pallas import tpu as pltpu

_B = 32
_D = 256
_H = 24
_W = 24
_HW = _H * _W


def _tile_kernel(p_ref, row_ref, col_ref, pose_ref, tile_ref, pemb_ref):
    r = jax.lax.broadcasted_iota(jnp.int32, (_D, _HW), 0)
    l = jax.lax.broadcasted_iota(jnp.int32, (_D, _HW), 1)
    sel_col = (r == l % _W + 1).astype(jnp.float32)
    sel_row = (r == l // _W + 1).astype(jnp.float32)
    dn = (((0,), (0,)), ((), ()))
    hp = jax.lax.Precision.HIGHEST
    tile_ref[0:_D, :] = jax.lax.dot_general(
        col_ref[...], sel_col, dn, precision=hp,
        preferred_element_type=jnp.float32)
    tile_ref[_D:2 * _D, :] = jax.lax.dot_general(
        row_ref[...], sel_row, dn, precision=hp,
        preferred_element_type=jnp.float32)

    onehot = (jax.lax.broadcasted_iota(jnp.int32, (8, _D), 1)
              == p_ref[0]).astype(jnp.float32)
    pv = jax.lax.dot_general(onehot, pose_ref[...], (((1,), (0,)), ((), ())),
                             precision=hp,
                             preferred_element_type=jnp.float32)
    row = pv[0:1, :]
    pemb_ref[...] = jnp.broadcast_to(
        jnp.concatenate([row, row], axis=1), (_B, 2 * _D))


def _fanout_kernel(tile_hbm, m_hbm, sem):
    copies = [pltpu.make_async_copy(tile_hbm, m_hbm.at[b], sem)
              for b in range(_B)]
    for c in copies:
        c.start()
    for c in copies:
        c.wait()


def kernel(x, row_W, col_W, pose_W, p):
    b, c, h, w = x.shape
    p_arr = jnp.asarray(p, dtype=jnp.int32).reshape((1,))
    tile, p_emb = pl.pallas_call(
        _tile_kernel,
        in_specs=[
            pl.BlockSpec(memory_space=pltpu.SMEM),
            pl.BlockSpec(memory_space=pltpu.MemorySpace.VMEM),
            pl.BlockSpec(memory_space=pltpu.MemorySpace.VMEM),
            pl.BlockSpec(memory_space=pltpu.MemorySpace.VMEM),
        ],
        out_specs=[
            pl.BlockSpec(memory_space=pltpu.MemorySpace.VMEM),
            pl.BlockSpec(memory_space=pltpu.MemorySpace.VMEM),
        ],
        out_shape=[
            jax.ShapeDtypeStruct((2 * _D, _HW), jnp.float32),
            jax.ShapeDtypeStruct((_B, 2 * _D), jnp.float32),
        ],
    )(p_arr, row_W, col_W, pose_W)
    m_flat = pl.pallas_call(
        _fanout_kernel,
        in_specs=[pl.BlockSpec(memory_space=pl.ANY)],
        out_specs=pl.BlockSpec(memory_space=pl.ANY),
        out_shape=jax.ShapeDtypeStruct((_B, 2 * _D, _HW), jnp.float32),
        scratch_shapes=[pltpu.SemaphoreType.DMA],
    )(tile)
    return (p_emb, m_flat.reshape(b, 2 * _D, h, w))


# speedup vs baseline: 18.3916x; 18.3916x over previous
import functools

import jax
import jax.numpy as jnp
from jax import lax
from jax.experimental import pallas as pl
from jax.experimental.pallas import tpu as pltpu
from jax.experimental.pallas import tpu_sc as plsc

_B = 32
_D = 256
_H = 24
_W = 24
_HW = _H * _W
_BB = 4

_NC = 2
_NS = 16


@functools.partial(
    pl.kernel,
    mesh=plsc.VectorSubcoreMesh(core_axis_name="c", subcore_axis_name="s"),
    out_type=jax.ShapeDtypeStruct((_B, 2, _D), jnp.float32),
    scratch_types=[
        pltpu.VMEM((1,), jnp.int32),
        pltpu.VMEM((1, _D), jnp.float32),
        pltpu.SemaphoreType.DMA,
    ],
)
def _pose_kernel(p_hbm, pose_hbm, out_hbm, idx_v, row_v, sem):
    c = lax.axis_index("c")
    s = lax.axis_index("s")
    b = c * (_B // _NC) + s
    pltpu.sync_copy(p_hbm, idx_v)
    pltpu.async_copy(pose_hbm.at[idx_v], row_v, sem).wait()
    cp0 = pltpu.async_copy(row_v, out_hbm.at[b, pl.ds(0, 1)], sem)
    cp1 = pltpu.async_copy(row_v, out_hbm.at[b, pl.ds(1, 1)], sem)
    cp0.wait()
    cp1.wait()


def _m_kernel(row_ref, col_ref, m_ref, scratch_ref):
    @pl.when(pl.program_id(0) == 0)
    def _():
        r = jax.lax.broadcasted_iota(jnp.int32, (_D, _HW), 0)
        l = jax.lax.broadcasted_iota(jnp.int32, (_D, _HW), 1)
        sel_col = (r == l % _W + 1).astype(jnp.float32)
        sel_row = (r == l // _W + 1).astype(jnp.float32)
        dn = (((0,), (0,)), ((), ()))
        hp = jax.lax.Precision.HIGHEST
        scratch_ref[0:_D, :] = jax.lax.dot_general(
            col_ref[...], sel_col, dn, precision=hp,
            preferred_element_type=jnp.float32)
        scratch_ref[_D:2 * _D, :] = jax.lax.dot_general(
            row_ref[...], sel_row, dn, precision=hp,
            preferred_element_type=jnp.float32)

    m_ref[...] = jnp.broadcast_to(scratch_ref[...][None, :, :],
                                  (_BB, 2 * _D, _HW))


def kernel(x, row_W, col_W, pose_W, p):
    b, c, h, w = x.shape
    p_arr = jnp.asarray(p, dtype=jnp.int32).reshape((1,))
    p_emb2 = _pose_kernel(p_arr, pose_W)
    m_flat = pl.pallas_call(
        _m_kernel,
        grid=(_B // _BB,),
        in_specs=[
            pl.BlockSpec((_D, _D), lambda i: (0, 0)),
            pl.BlockSpec((_D, _D), lambda i: (0, 0)),
        ],
        out_specs=pl.BlockSpec((_BB, 2 * _D, _HW), lambda i: (i, 0, 0)),
        out_shape=jax.ShapeDtypeStruct((_B, 2 * _D, _HW), jnp.float32),
        scratch_shapes=[pltpu.VMEM((2 * _D, _HW), jnp.float32)],
    )(row_W, col_W)
    return (p_emb2.reshape(b, 2 * _D), m_flat.reshape(b, 2 * _D, h, w))
